# Initial kernel scaffold; baseline (speedup 1.0000x reference)
import jax
import jax.numpy as jnp
from jax.experimental import pallas as pl


def kernel(x, W, temperature):
    raise NotImplementedError("write your pallas kernel here")


# speedup vs baseline: 3.0823x; 3.0823x over previous
import jax
import jax.numpy as jnp
from jax.experimental import pallas as pl
from jax.experimental.pallas import tpu as pltpu

N_EXPERTS = 16
TOP_K = 2
D_MODEL = 2048
N_TOKENS = 16384

BLK = 1024


def _router_body(t_ref, x_ref, w_ref, rm_ref, idx_ref):
    inv_t = 1.0 / t_ref[0]
    logits = jax.lax.dot_general(
        x_ref[...], w_ref[...],
        dimension_numbers=(((1,), (1,)), ((), ())),
        preferred_element_type=jnp.float32,
    ) * inv_t
    e_iota = jax.lax.broadcasted_iota(jnp.int32, logits.shape, 1)
    big = jnp.int32(N_EXPERTS)
    m0 = jnp.max(logits, axis=1, keepdims=True)
    i0 = jnp.min(jnp.where(logits == m0, e_iota, big), axis=1, keepdims=True)
    masked = jnp.where(e_iota == i0, -jnp.inf, logits)
    m1 = jnp.max(masked, axis=1, keepdims=True)
    i1 = jnp.min(jnp.where(masked == m1, e_iota, big), axis=1, keepdims=True)
    e = jnp.exp(m1 - m0)
    w0 = 1.0 / (1.0 + e)
    w1 = e * w0
    rm_ref[...] = jnp.where(e_iota == i0, w0,
                            jnp.where(e_iota == i1, w1, jnp.float32(0.0)))
    idx_ref[...] = jnp.concatenate([i0, i1], axis=1)


def kernel(x, W, temperature):
    t = jnp.asarray(temperature, jnp.float32).reshape(1)
    grid = (N_TOKENS // BLK,)
    rm, idx = pl.pallas_call(
        _router_body,
        grid=grid,
        in_specs=[
            pl.BlockSpec(memory_space=pltpu.SMEM),
            pl.BlockSpec((BLK, D_MODEL), lambda i: (i, 0)),
            pl.BlockSpec((N_EXPERTS, D_MODEL), lambda i: (0, 0)),
        ],
        out_specs=[
            pl.BlockSpec((BLK, N_EXPERTS), lambda i: (i, 0)),
            pl.BlockSpec((BLK, TOP_K), lambda i: (i, 0)),
        ],
        out_shape=[
            jax.ShapeDtypeStruct((N_TOKENS, N_EXPERTS), jnp.float32),
            jax.ShapeDtypeStruct((N_TOKENS, TOP_K), jnp.int32),
        ],
        compiler_params=pltpu.CompilerParams(
            dimension_semantics=("arbitrary",),
        ),
    )(t, x, W)
    return (rm, idx)


# speedup vs baseline: 3.2132x; 1.0425x over previous
import jax
import jax.numpy as jnp
from jax.experimental import pallas as pl
from jax.experimental.pallas import tpu as pltpu

N_EXPERTS = 16
TOP_K = 2
D_MODEL = 2048
N_TOKENS = 16384

BLK = 2048


def _router_body(t_ref, x_ref, w_ref, rm_ref, idx_ref):
    inv_t = 1.0 / t_ref[0]
    logits = jax.lax.dot_general(
        x_ref[...], w_ref[...],
        dimension_numbers=(((1,), (1,)), ((), ())),
        preferred_element_type=jnp.float32,
    ) * inv_t
    e_iota = jax.lax.broadcasted_iota(jnp.int32, logits.shape, 1)
    big = jnp.int32(N_EXPERTS)
    m0 = jnp.max(logits, axis=1, keepdims=True)
    i0 = jnp.min(jnp.where(logits == m0, e_iota, big), axis=1, keepdims=True)
    masked = jnp.where(e_iota == i0, -jnp.inf, logits)
    m1 = jnp.max(masked, axis=1, keepdims=True)
    i1 = jnp.min(jnp.where(masked == m1, e_iota, big), axis=1, keepdims=True)
    e = jnp.exp(m1 - m0)
    w0 = 1.0 / (1.0 + e)
    w1 = e * w0
    rm_ref[...] = jnp.where(e_iota == i0, w0,
                            jnp.where(e_iota == i1, w1, jnp.float32(0.0)))
    idx_ref[...] = jnp.concatenate([i0, i1], axis=1)


def kernel(x, W, temperature):
    t = jnp.asarray(temperature, jnp.float32).reshape(1)
    grid = (N_TOKENS // BLK,)
    rm, idx = pl.pallas_call(
        _router_body,
        grid=grid,
        in_specs=[
            pl.BlockSpec(memory_space=pltpu.SMEM),
            pl.BlockSpec((BLK, D_MODEL), lambda i: (i, 0)),
            pl.BlockSpec((N_EXPERTS, D_MODEL), lambda i: (0, 0)),
        ],
        out_specs=[
            pl.BlockSpec((BLK, N_EXPERTS), lambda i: (i, 0)),
            pl.BlockSpec((BLK, TOP_K), lambda i: (i, 0)),
        ],
        out_shape=[
            jax.ShapeDtypeStruct((N_TOKENS, N_EXPERTS), jnp.float32),
            jax.ShapeDtypeStruct((N_TOKENS, TOP_K), jnp.int32),
        ],
        compiler_params=pltpu.CompilerParams(
            dimension_semantics=("arbitrary",),
        ),
    )(t, x, W)
    return (rm, idx)
